# speedup vs baseline: 2.3999x; 2.3999x over previous
import jax
import jax.numpy as jnp
from jax.experimental import pallas as pl
from jax.experimental.pallas import tpu as pltpu

NCLS = 50
BATCH = 64
NPTS = 4096
SUB = 8
LANE = NPTS // SUB
BPB = 4
GRID = BATCH // BPB


def _body(logits_ref, labels_ref, inter_ref, union_ref,
          acc_i_ref, acc_p_ref, acc_l_ref):
    step = pl.program_id(0)

    @pl.when(step == 0)
    def _init():
        acc_i_ref[...] = jnp.zeros_like(acc_i_ref)
        acc_p_ref[...] = jnp.zeros_like(acc_p_ref)
        acc_l_ref[...] = jnp.zeros_like(acc_l_ref)

    preds = []
    labs = []
    for b in range(BPB):
        best = logits_ref[b, 0]
        idx = jnp.zeros((SUB, LANE), jnp.int32)
        for c in range(1, NCLS):
            v = logits_ref[b, c]
            gt = v > best
            best = jnp.where(gt, v, best)
            idx = jnp.where(gt, c, idx)
        preds.append(idx)
        labs.append(labels_ref[b])

    one = jnp.float32(1.0)
    zero = jnp.float32(0.0)
    for c in range(NCLS):
        pi = zero
        pp = zero
        ll = zero
        for b in range(BPB):
            e1 = preds[b] == c
            e2 = labs[b] == c
            im = e1 & e2
            pp = pp + jnp.where(e1, one, zero)
            ll = ll + jnp.where(e2, one, zero)
            pi = pi + jnp.where(im, one, zero)
        acc_i_ref[c] = acc_i_ref[c] + pi
        acc_p_ref[c] = acc_p_ref[c] + pp
        acc_l_ref[c] = acc_l_ref[c] + ll

    @pl.when(step == GRID - 1)
    def _fin():
        si = jnp.sum(acc_i_ref[...], axis=(1, 2))
        sp = jnp.sum(acc_p_ref[...], axis=(1, 2))
        sl = jnp.sum(acc_l_ref[...], axis=(1, 2))
        inter_ref[...] = jnp.broadcast_to(si[:, None], (NCLS, 128))
        union_ref[...] = jnp.broadcast_to((sp + sl - si)[:, None], (NCLS, 128))


@jax.jit
def kernel(seg_logits, seg_labels):
    logits = seg_logits.reshape(BATCH, NCLS, SUB, LANE)
    labels = seg_labels.reshape(BATCH, SUB, LANE)
    out = pl.pallas_call(
        _body,
        grid=(GRID,),
        in_specs=[
            pl.BlockSpec((BPB, NCLS, SUB, LANE), lambda i: (i, 0, 0, 0)),
            pl.BlockSpec((BPB, SUB, LANE), lambda i: (i, 0, 0)),
        ],
        out_specs=[
            pl.BlockSpec((NCLS, 128), lambda i: (0, 0)),
            pl.BlockSpec((NCLS, 128), lambda i: (0, 0)),
        ],
        out_shape=[
            jax.ShapeDtypeStruct((NCLS, 128), jnp.float32),
            jax.ShapeDtypeStruct((NCLS, 128), jnp.float32),
        ],
        scratch_shapes=[
            pltpu.VMEM((NCLS, SUB, LANE), jnp.float32),
            pltpu.VMEM((NCLS, SUB, LANE), jnp.float32),
            pltpu.VMEM((NCLS, SUB, LANE), jnp.float32),
        ],
    )(logits, labels)
    return out[0][:, 0], out[1][:, 0]


# speedup vs baseline: 2.9470x; 1.2279x over previous
import functools

import jax
import jax.numpy as jnp
from jax import lax
from jax.experimental import pallas as pl
from jax.experimental.pallas import tpu as pltpu
from jax.experimental.pallas import tpu_sc as plsc

NCLS = 50
BATCH = 64
NPTS = 4096
BPB = 8
TC_GRID = BATCH // BPB

NWORK = 32
ROWS_PER_W = BATCH // NWORK
HWORDS = 192


def _argmax_body(logits_ref, pred_ref):
    for b in range(BPB):
        best = logits_ref[b, 0:8]
        gidx = jnp.zeros((8, NPTS), jnp.int32)
        for g in range(1, 6):
            v = logits_ref[b, 8 * g:8 * g + 8]
            gt = v > best
            best = jnp.where(gt, v, best)
            gidx = jnp.where(gt, g, gidx)
        sub = lax.broadcasted_iota(jnp.int32, (8, NPTS), 0)
        cls8 = gidx * 8 + sub
        m = jnp.max(best, axis=0, keepdims=True)
        cand = jnp.where(best == m, cls8, NCLS + 77)
        pcls = jnp.min(cand, axis=0, keepdims=True)
        pval = m
        for c in (48, 49):
            v = logits_ref[b, c:c + 1]
            gt = v > pval
            pval = jnp.where(gt, v, pval)
            pcls = jnp.where(gt, c, pcls)
        pred_ref[pl.ds(b, 1), :] = pcls


def _tc_argmax(seg_logits):
    return pl.pallas_call(
        _argmax_body,
        grid=(TC_GRID,),
        in_specs=[pl.BlockSpec((BPB, NCLS, NPTS), lambda i: (i, 0, 0))],
        out_specs=pl.BlockSpec((BPB, NPTS), lambda i: (i, 0)),
        out_shape=jax.ShapeDtypeStruct((BATCH, NPTS), jnp.int32),
    )(seg_logits)


@functools.partial(
    pl.kernel,
    out_type=jax.ShapeDtypeStruct((NWORK, HWORDS), jnp.float32),
    mesh=plsc.VectorSubcoreMesh(core_axis_name="c", subcore_axis_name="s"),
    compiler_params=pltpu.CompilerParams(needs_layout_passes=False),
    scratch_types=[
        pltpu.VMEM((ROWS_PER_W, NPTS), jnp.int32),
        pltpu.VMEM((ROWS_PER_W, NPTS), jnp.int32),
        pltpu.VMEM((16 * HWORDS,), jnp.float32),
        pltpu.VMEM((HWORDS,), jnp.float32),
    ],
)
def _sc_hist(pred_hbm, lab_hbm, out_hbm, pv, lv, lhist, hist):
    cid = lax.axis_index("c")
    sid = lax.axis_index("s")
    w = sid * 2 + cid
    base = w * ROWS_PER_W
    pltpu.sync_copy(pred_hbm.at[pl.ds(base, ROWS_PER_W)], pv)
    pltpu.sync_copy(lab_hbm.at[pl.ds(base, ROWS_PER_W)], lv)

    zeros16 = jnp.zeros((16,), jnp.float32)
    for j in range(16 * HWORDS // 16):
        lhist[pl.ds(j * 16, 16)] = zeros16

    ones = jnp.ones((16,), jnp.float32)
    lbase = lax.iota(jnp.int32, 16) * HWORDS

    def chunk(r):
        def body(k, carry):
            off = k * 16
            p = pv[r, pl.ds(off, 16)]
            l = lv[r, pl.ds(off, 16)]
            mval = jnp.where(p == l, 1.0, 0.0).astype(jnp.float32)
            ip = lbase + p
            plsc.addupdate_scatter(lhist, [lbase + (l + 128)], ones)
            plsc.addupdate_scatter(lhist, [ip], mval)
            plsc.addupdate_scatter(lhist, [ip + 64], ones)
            return carry
        lax.fori_loop(0, NPTS // 16, body, 0)

    for r in range(ROWS_PER_W):
        chunk(r)

    for j in range(HWORDS // 16):
        acc = zeros16
        for t in range(16):
            acc = acc + lhist[pl.ds(t * HWORDS + j * 16, 16)]
        hist[pl.ds(j * 16, 16)] = acc

    pltpu.sync_copy(hist, out_hbm.at[w])


@jax.jit
def kernel(seg_logits, seg_labels):
    pred = _tc_argmax(seg_logits)
    part = _sc_hist(pred, seg_labels)
    res = jnp.sum(part, axis=0)
    inter = res[0:NCLS]
    union = res[64:64 + NCLS] + res[128:128 + NCLS] - inter
    return inter, union


# speedup vs baseline: 6.4583x; 2.1915x over previous
import functools

import jax
import jax.numpy as jnp
from jax import lax
from jax.experimental import pallas as pl
from jax.experimental.pallas import tpu as pltpu
from jax.experimental.pallas import tpu_sc as plsc

NCLS = 50
BATCH = 64
NPTS = 4096
BPB = 8
TC_GRID = BATCH // BPB

NWORK = 32
ROWS_PER_W = BATCH // NWORK
HWORDS = 192


def _argmax_body(logits_ref, pred_ref):
    best = logits_ref[0]
    idx = jnp.zeros((BPB, NPTS), jnp.int32)
    for c in range(1, NCLS):
        v = logits_ref[c]
        gt = v > best
        best = jnp.where(gt, v, best)
        idx = jnp.where(gt, c, idx)
    pred_ref[...] = idx


def _tc_argmax(logits_cmajor):
    return pl.pallas_call(
        _argmax_body,
        grid=(TC_GRID,),
        in_specs=[pl.BlockSpec((NCLS, BPB, NPTS), lambda i: (0, i, 0))],
        out_specs=pl.BlockSpec((BPB, NPTS), lambda i: (i, 0)),
        out_shape=jax.ShapeDtypeStruct((BATCH, NPTS), jnp.int32),
    )(logits_cmajor)


@functools.partial(
    pl.kernel,
    out_type=jax.ShapeDtypeStruct((NWORK, HWORDS), jnp.float32),
    mesh=plsc.VectorSubcoreMesh(core_axis_name="c", subcore_axis_name="s"),
    compiler_params=pltpu.CompilerParams(needs_layout_passes=False),
    scratch_types=[
        pltpu.VMEM((ROWS_PER_W, NPTS), jnp.int32),
        pltpu.VMEM((ROWS_PER_W, NPTS), jnp.int32),
        pltpu.VMEM((16 * HWORDS,), jnp.float32),
        pltpu.VMEM((HWORDS,), jnp.float32),
    ],
)
def _sc_hist(pred_hbm, lab_hbm, out_hbm, pv, lv, lhist, hist):
    cid = lax.axis_index("c")
    sid = lax.axis_index("s")
    w = sid * 2 + cid
    base = w * ROWS_PER_W
    pltpu.sync_copy(pred_hbm.at[pl.ds(base, ROWS_PER_W)], pv)
    pltpu.sync_copy(lab_hbm.at[pl.ds(base, ROWS_PER_W)], lv)

    zeros16 = jnp.zeros((16,), jnp.float32)
    for j in range(16 * HWORDS // 16):
        lhist[pl.ds(j * 16, 16)] = zeros16

    ones = jnp.ones((16,), jnp.float32)
    lbase = lax.iota(jnp.int32, 16) * HWORDS

    def chunk(r):
        def body(k, carry):
            off = k * 16
            p = pv[r, pl.ds(off, 16)]
            l = lv[r, pl.ds(off, 16)]
            mval = jnp.where(p == l, 1.0, 0.0).astype(jnp.float32)
            ip = lbase + p
            plsc.addupdate_scatter(lhist, [lbase + (l + 128)], ones)
            plsc.addupdate_scatter(lhist, [ip], mval)
            plsc.addupdate_scatter(lhist, [ip + 64], ones)
            return carry
        lax.fori_loop(0, NPTS // 16, body, 0)

    for r in range(ROWS_PER_W):
        chunk(r)

    for j in range(HWORDS // 16):
        acc = zeros16
        for t in range(16):
            acc = acc + lhist[pl.ds(t * HWORDS + j * 16, 16)]
        hist[pl.ds(j * 16, 16)] = acc

    pltpu.sync_copy(hist, out_hbm.at[w])


@jax.jit
def kernel(seg_logits, seg_labels):
    pred = _tc_argmax(jnp.transpose(seg_logits, (1, 0, 2)))
    part = _sc_hist(pred, seg_labels)
    res = jnp.sum(part, axis=0)
    inter = res[0:NCLS]
    union = res[64:64 + NCLS] + res[128:128 + NCLS] - inter
    return inter, union
